# baseline (speedup 1.0000x reference)
import functools

import jax
import jax.numpy as jnp
from jax import lax
from jax.experimental import pallas as pl
from jax.experimental.pallas import tpu as pltpu

_D = 768
_E = 64
_K = 8
_LBW = 0.01

_BT = 512


def _router_block(x_ref, w_ref, probs_ref, wts_ref, idx_ref, loss_ref,
                  acc_p, acc_c, *, nsteps, num_tokens):
    step = pl.program_id(0)

    xb = x_ref[...]
    w = w_ref[...]
    logits = lax.dot_general(xb, w, (((1,), (1,)), ((), ())),
                             preferred_element_type=jnp.float32)

    m = jnp.max(logits, axis=1, keepdims=True)
    ex = jnp.exp(logits - m)
    s = jnp.sum(ex, axis=1, keepdims=True)
    probs = ex / s
    probs_ref[...] = probs

    lane = lax.broadcasted_iota(jnp.int32, (_BT, _E), 1)
    keys = lax.bitcast_convert_type(probs, jnp.int32)
    sel = jnp.zeros((_BT, _E), jnp.float32)
    vals, idxs = [], []
    for _ in range(_K):
        mk = jnp.max(keys, axis=1, keepdims=True)
        ik = jnp.min(jnp.where(keys == mk, lane, _E), axis=1, keepdims=True)
        chosen = lane == ik
        keys = jnp.where(chosen, -1, keys)
        sel = sel + chosen.astype(jnp.float32)
        vals.append(lax.bitcast_convert_type(mk, jnp.float32))
        idxs.append(ik)

    w8 = jnp.concatenate(vals, axis=1)
    wts_ref[...] = w8 / jnp.sum(w8, axis=1, keepdims=True)
    idx_ref[...] = jnp.concatenate(idxs, axis=1)

    psum = jnp.sum(probs, axis=0, keepdims=True)
    csum = jnp.sum(sel, axis=0, keepdims=True)

    @pl.when(step == 0)
    def _init():
        acc_p[...] = psum
        acc_c[...] = csum

    @pl.when(step > 0)
    def _acc():
        acc_p[...] += psum
        acc_c[...] += csum

    @pl.when(step == nsteps - 1)
    def _fin():
        scale = (_LBW * _E) / (float(num_tokens) * float(num_tokens))
        loss_ref[...] = scale * jnp.sum(acc_p[...] * acc_c[...],
                                        keepdims=True).reshape(1, 1)


def kernel(x, W):
    B, S, D = x.shape
    T = B * S
    xf = x.reshape(T, D)
    nsteps = T // _BT
    body = functools.partial(_router_block, nsteps=nsteps, num_tokens=T)
    probs, wts, idx, loss = pl.pallas_call(
        body,
        grid=(nsteps,),
        in_specs=[
            pl.BlockSpec((_BT, D), lambda i: (i, 0)),
            pl.BlockSpec((_E, D), lambda i: (0, 0)),
        ],
        out_specs=[
            pl.BlockSpec((_BT, _E), lambda i: (i, 0)),
            pl.BlockSpec((_BT, _K), lambda i: (i, 0)),
            pl.BlockSpec((_BT, _K), lambda i: (i, 0)),
            pl.BlockSpec((1, 1), lambda i: (0, 0)),
        ],
        out_shape=[
            jax.ShapeDtypeStruct((T, _E), jnp.float32),
            jax.ShapeDtypeStruct((T, _K), jnp.float32),
            jax.ShapeDtypeStruct((T, _K), jnp.int32),
            jax.ShapeDtypeStruct((1, 1), jnp.float32),
        ],
        scratch_shapes=[
            pltpu.VMEM((1, _E), jnp.float32),
            pltpu.VMEM((1, _E), jnp.float32),
        ],
        compiler_params=pltpu.CompilerParams(
            dimension_semantics=("arbitrary",)),
    )(xf, W)
    return (wts.reshape(B, S, _K), idx.reshape(B, S, _K), loss[0, 0],
            probs.reshape(B, S, _E))


# speedup vs baseline: 1.8420x; 1.8420x over previous
import functools

import jax
import jax.numpy as jnp
from jax import lax
from jax.experimental import pallas as pl
from jax.experimental.pallas import tpu as pltpu

_D = 768
_E = 64
_K = 8
_LBW = 0.01

_BT = 512


def _router_block(x_ref, w_ref, probs_ref, wts_ref, idx_ref, loss_ref,
                  acc_p, acc_c, *, nsteps, num_tokens):
    step = pl.program_id(0)

    xb = x_ref[...]
    w = w_ref[...]
    logits = lax.dot_general(w, xb, (((1,), (1,)), ((), ())),
                             preferred_element_type=jnp.float32)

    m = jnp.max(logits, axis=0, keepdims=True)
    ex = jnp.exp(logits - m)
    s = jnp.sum(ex, axis=0, keepdims=True)
    probs = ex * (1.0 / s)
    probs_ref[...] = probs.T

    row = lax.broadcasted_iota(jnp.int32, (_E, _BT), 0)
    keys = lax.bitcast_convert_type(probs, jnp.int32)
    sel = jnp.zeros((_E, _BT), jnp.float32)
    vals, idxs = [], []
    for _ in range(_K):
        mk = jnp.max(keys, axis=0, keepdims=True)
        ik = jnp.min(jnp.where(keys == mk, row, _E), axis=0, keepdims=True)
        chosen = row == ik
        keys = jnp.where(chosen, -1, keys)
        sel = sel + chosen.astype(jnp.float32)
        vals.append(lax.bitcast_convert_type(mk, jnp.float32))
        idxs.append(ik.astype(jnp.float32))

    w8 = jnp.concatenate(vals, axis=0)
    w8 = w8 * (1.0 / jnp.sum(w8, axis=0, keepdims=True))
    wts_ref[...] = w8.T
    i8 = jnp.concatenate(idxs, axis=0)
    idx_ref[...] = i8.T.astype(jnp.int32)

    psum = jnp.sum(probs, axis=1, keepdims=True)
    csum = jnp.sum(sel, axis=1, keepdims=True)

    @pl.when(step == 0)
    def _init():
        acc_p[...] = psum
        acc_c[...] = csum

    @pl.when(step > 0)
    def _acc():
        acc_p[...] += psum
        acc_c[...] += csum

    @pl.when(step == nsteps - 1)
    def _fin():
        scale = (_LBW * _E) / (float(num_tokens) * float(num_tokens))
        loss_ref[...] = scale * jnp.sum(acc_p[...] * acc_c[...],
                                        keepdims=True).reshape(1, 1)


def kernel(x, W):
    B, S, D = x.shape
    T = B * S
    xf = x.reshape(T, D)
    nsteps = T // _BT
    body = functools.partial(_router_block, nsteps=nsteps, num_tokens=T)
    probs, wts, idx, loss = pl.pallas_call(
        body,
        grid=(nsteps,),
        in_specs=[
            pl.BlockSpec((_BT, D), lambda i: (i, 0)),
            pl.BlockSpec((_E, D), lambda i: (0, 0)),
        ],
        out_specs=[
            pl.BlockSpec((_BT, _E), lambda i: (i, 0)),
            pl.BlockSpec((_BT, _K), lambda i: (i, 0)),
            pl.BlockSpec((_BT, _K), lambda i: (i, 0)),
            pl.BlockSpec((1, 1), lambda i: (0, 0)),
        ],
        out_shape=[
            jax.ShapeDtypeStruct((T, _E), jnp.float32),
            jax.ShapeDtypeStruct((T, _K), jnp.float32),
            jax.ShapeDtypeStruct((T, _K), jnp.int32),
            jax.ShapeDtypeStruct((1, 1), jnp.float32),
        ],
        scratch_shapes=[
            pltpu.VMEM((_E, 1), jnp.float32),
            pltpu.VMEM((_E, 1), jnp.float32),
        ],
        compiler_params=pltpu.CompilerParams(
            dimension_semantics=("arbitrary",)),
    )(xf, W)
    return (wts.reshape(B, S, _K), idx.reshape(B, S, _K), loss[0, 0],
            probs.reshape(B, S, _E))


# speedup vs baseline: 2.2852x; 1.2407x over previous
import functools

import jax
import jax.numpy as jnp
from jax import lax
from jax.experimental import pallas as pl
from jax.experimental.pallas import tpu as pltpu

_D = 768
_E = 64
_K = 8
_LBW = 0.01

_BT = 1024


def _router_block(x_ref, w_ref, probs_ref, wts_ref, idx_ref, loss_ref,
                  acc_p, acc_c, *, nsteps, num_tokens):
    step = pl.program_id(0)

    xb = x_ref[...]
    w = w_ref[...]
    logits = lax.dot_general(w, xb, (((1,), (1,)), ((), ())),
                             preferred_element_type=jnp.float32)

    m = jnp.max(logits, axis=0, keepdims=True)
    ex = jnp.exp(logits - m)
    s = jnp.sum(ex, axis=0, keepdims=True)
    probs = ex * (1.0 / s)
    probs_ref[...] = probs.T

    row = lax.broadcasted_iota(jnp.int32, (_E, _BT), 0)
    keys = lax.bitcast_convert_type(probs, jnp.int32)
    vals, idxs = [], []
    for _ in range(_K):
        mk = jnp.max(keys, axis=0, keepdims=True)
        ik = jnp.min(jnp.where(keys == mk, row, _E), axis=0, keepdims=True)
        keys = jnp.where(row == ik, -1, keys)
        vals.append(lax.bitcast_convert_type(mk, jnp.float32))
        idxs.append(ik.astype(jnp.float32))
    sel = (keys < 0).astype(jnp.float32)

    w8 = jnp.concatenate(vals, axis=0)
    w8 = w8 * (1.0 / jnp.sum(w8, axis=0, keepdims=True))
    wts_ref[...] = w8.T
    i8 = jnp.concatenate(idxs, axis=0)
    idx_ref[...] = i8.T.astype(jnp.int32)

    psum = jnp.sum(probs, axis=1, keepdims=True)
    csum = jnp.sum(sel, axis=1, keepdims=True)

    @pl.when(step == 0)
    def _init():
        acc_p[...] = psum
        acc_c[...] = csum

    @pl.when(step > 0)
    def _acc():
        acc_p[...] += psum
        acc_c[...] += csum

    @pl.when(step == nsteps - 1)
    def _fin():
        scale = (_LBW * _E) / (float(num_tokens) * float(num_tokens))
        loss_ref[...] = scale * jnp.sum(acc_p[...] * acc_c[...],
                                        keepdims=True).reshape(1, 1)


def kernel(x, W):
    B, S, D = x.shape
    T = B * S
    xf = x.reshape(T, D)
    nsteps = T // _BT
    body = functools.partial(_router_block, nsteps=nsteps, num_tokens=T)
    probs, wts, idx, loss = pl.pallas_call(
        body,
        grid=(nsteps,),
        in_specs=[
            pl.BlockSpec((_BT, D), lambda i: (i, 0)),
            pl.BlockSpec((_E, D), lambda i: (0, 0)),
        ],
        out_specs=[
            pl.BlockSpec((_BT, _E), lambda i: (i, 0)),
            pl.BlockSpec((_BT, _K), lambda i: (i, 0)),
            pl.BlockSpec((_BT, _K), lambda i: (i, 0)),
            pl.BlockSpec((1, 1), lambda i: (0, 0)),
        ],
        out_shape=[
            jax.ShapeDtypeStruct((T, _E), jnp.float32),
            jax.ShapeDtypeStruct((T, _K), jnp.float32),
            jax.ShapeDtypeStruct((T, _K), jnp.int32),
            jax.ShapeDtypeStruct((1, 1), jnp.float32),
        ],
        scratch_shapes=[
            pltpu.VMEM((_E, 1), jnp.float32),
            pltpu.VMEM((_E, 1), jnp.float32),
        ],
        compiler_params=pltpu.CompilerParams(
            dimension_semantics=("arbitrary",)),
    )(xf, W)
    return (wts.reshape(B, S, _K), idx.reshape(B, S, _K), loss[0, 0],
            probs.reshape(B, S, _E))


# speedup vs baseline: 2.4082x; 1.0538x over previous
import functools

import jax
import jax.numpy as jnp
from jax import lax
from jax.experimental import pallas as pl
from jax.experimental.pallas import tpu as pltpu

_D = 768
_E = 64
_K = 8
_LBW = 0.01

_BT = 1024


def _router_block(x_ref, w_ref, probs_ref, wts_ref, idx_ref, loss_ref,
                  acc_p, acc_c, *, nsteps, num_tokens):
    step = pl.program_id(0) * pl.num_programs(1) + pl.program_id(1)

    xb = x_ref[0]
    w = w_ref[...]
    logits = lax.dot_general(w, xb, (((1,), (1,)), ((), ())),
                             preferred_element_type=jnp.float32)

    m = jnp.max(logits, axis=0, keepdims=True)
    ex = jnp.exp(logits - m)
    s = jnp.sum(ex, axis=0, keepdims=True)
    probs = ex * (1.0 / s)
    probs_ref[0] = probs.T

    row = lax.broadcasted_iota(jnp.int32, (_E, _BT), 0)
    keys = lax.bitcast_convert_type(probs, jnp.int32)
    vals, idxs = [], []
    for _ in range(_K):
        mk = jnp.max(keys, axis=0, keepdims=True)
        ik = jnp.min(jnp.where(keys == mk, row, _E), axis=0, keepdims=True)
        keys = jnp.where(row == ik, -1, keys)
        vals.append(lax.bitcast_convert_type(mk, jnp.float32))
        idxs.append(ik.astype(jnp.float32))
    sel = (keys < 0).astype(jnp.float32)

    w8 = jnp.concatenate(vals, axis=0)
    w8 = w8 * (1.0 / jnp.sum(w8, axis=0, keepdims=True))
    wts_ref[0] = w8.T
    i8 = jnp.concatenate(idxs, axis=0)
    idx_ref[0] = i8.T.astype(jnp.int32)

    psum = jnp.sum(probs, axis=1, keepdims=True)
    csum = jnp.sum(sel, axis=1, keepdims=True)

    @pl.when(step == 0)
    def _init():
        acc_p[...] = psum
        acc_c[...] = csum

    @pl.when(step > 0)
    def _acc():
        acc_p[...] += psum
        acc_c[...] += csum

    @pl.when(step == nsteps - 1)
    def _fin():
        scale = (_LBW * _E) / (float(num_tokens) * float(num_tokens))
        loss_ref[...] = scale * jnp.sum(acc_p[...] * acc_c[...],
                                        keepdims=True).reshape(1, 1)


def kernel(x, W):
    B, S, D = x.shape
    T = B * S
    nsb = S // _BT
    nsteps = B * nsb
    body = functools.partial(_router_block, nsteps=nsteps, num_tokens=T)
    probs, wts, idx, loss = pl.pallas_call(
        body,
        grid=(B, nsb),
        in_specs=[
            pl.BlockSpec((1, _BT, D), lambda b, i: (b, i, 0)),
            pl.BlockSpec((_E, D), lambda b, i: (0, 0)),
        ],
        out_specs=[
            pl.BlockSpec((1, _BT, _E), lambda b, i: (b, i, 0)),
            pl.BlockSpec((1, _BT, _K), lambda b, i: (b, i, 0)),
            pl.BlockSpec((1, _BT, _K), lambda b, i: (b, i, 0)),
            pl.BlockSpec((1, 1), lambda b, i: (0, 0)),
        ],
        out_shape=[
            jax.ShapeDtypeStruct((B, S, _E), jnp.float32),
            jax.ShapeDtypeStruct((B, S, _K), jnp.float32),
            jax.ShapeDtypeStruct((B, S, _K), jnp.int32),
            jax.ShapeDtypeStruct((1, 1), jnp.float32),
        ],
        scratch_shapes=[
            pltpu.VMEM((_E, 1), jnp.float32),
            pltpu.VMEM((_E, 1), jnp.float32),
        ],
        compiler_params=pltpu.CompilerParams(
            dimension_semantics=("arbitrary", "arbitrary")),
    )(x, W)
    return (wts, idx, loss[0, 0], probs)


# speedup vs baseline: 3.3646x; 1.3972x over previous
import functools

import jax
import jax.numpy as jnp
from jax import lax
from jax.experimental import pallas as pl
from jax.experimental.pallas import tpu as pltpu

_D = 768
_E = 64
_K = 8
_LBW = 0.01

_BT = 1024


def _router_block(x_ref, w_ref, probs_ref, wts_ref, idx_ref, loss_ref,
                  acc_p, acc_c, *, nsteps, num_tokens):
    step = pl.program_id(0) * pl.num_programs(1) + pl.program_id(1)

    xb = x_ref[0]
    w = w_ref[...]
    logits = lax.dot_general(w, xb, (((1,), (1,)), ((), ())),
                             preferred_element_type=jnp.float32)

    m = jnp.max(logits, axis=0, keepdims=True)
    ex = jnp.exp(logits - m)
    s = jnp.sum(ex, axis=0, keepdims=True)
    probs = ex * (1.0 / s)
    probs_ref[0] = probs.T

    row = lax.broadcasted_iota(jnp.int32, (_E, _BT), 0)
    keys = lax.bitcast_convert_type(probs, jnp.int32)
    vals, idxs = [], []
    for _ in range(_K):
        mk = jnp.max(keys, axis=0, keepdims=True)
        ik = jnp.min(jnp.where(keys == mk, row, _E), axis=0, keepdims=True)
        keys = jnp.where(row == ik, -1, keys)
        vals.append(lax.bitcast_convert_type(mk, jnp.float32))
        idxs.append(ik.astype(jnp.float32))
    sel = (keys < 0).astype(jnp.float32)

    w8 = jnp.concatenate(vals, axis=0)
    w8 = w8 * (1.0 / jnp.sum(w8, axis=0, keepdims=True))
    wts_ref[0] = w8
    i8 = jnp.concatenate(idxs, axis=0)
    idx_ref[0] = i8.astype(jnp.int32)

    psum = jnp.sum(probs, axis=1, keepdims=True)
    csum = jnp.sum(sel, axis=1, keepdims=True)

    @pl.when(step == 0)
    def _init():
        acc_p[...] = psum
        acc_c[...] = csum

    @pl.when(step > 0)
    def _acc():
        acc_p[...] += psum
        acc_c[...] += csum

    @pl.when(step == nsteps - 1)
    def _fin():
        scale = (_LBW * _E) / (float(num_tokens) * float(num_tokens))
        loss_ref[...] = scale * jnp.sum(acc_p[...] * acc_c[...],
                                        keepdims=True).reshape(1, 1)


def kernel(x, W):
    B, S, D = x.shape
    T = B * S
    nsb = S // _BT
    nsteps = B * nsb
    body = functools.partial(_router_block, nsteps=nsteps, num_tokens=T)
    probs, wts, idx, loss = pl.pallas_call(
        body,
        grid=(B, nsb),
        in_specs=[
            pl.BlockSpec((1, _BT, D), lambda b, i: (b, i, 0)),
            pl.BlockSpec((_E, D), lambda b, i: (0, 0)),
        ],
        out_specs=[
            pl.BlockSpec((1, _BT, _E), lambda b, i: (b, i, 0)),
            pl.BlockSpec((1, _K, _BT), lambda b, i: (b, 0, i)),
            pl.BlockSpec((1, _K, _BT), lambda b, i: (b, 0, i)),
            pl.BlockSpec((1, 1), lambda b, i: (0, 0)),
        ],
        out_shape=[
            jax.ShapeDtypeStruct((B, S, _E), jnp.float32),
            jax.ShapeDtypeStruct((B, _K, S), jnp.float32),
            jax.ShapeDtypeStruct((B, _K, S), jnp.int32),
            jax.ShapeDtypeStruct((1, 1), jnp.float32),
        ],
        scratch_shapes=[
            pltpu.VMEM((_E, 1), jnp.float32),
            pltpu.VMEM((_E, 1), jnp.float32),
        ],
        compiler_params=pltpu.CompilerParams(
            dimension_semantics=("arbitrary", "arbitrary")),
    )(x, W)
    return (wts.transpose(0, 2, 1), idx.transpose(0, 2, 1), loss[0, 0], probs)


# speedup vs baseline: 3.8569x; 1.1463x over previous
import functools

import jax
import jax.numpy as jnp
from jax import lax
from jax.experimental import pallas as pl
from jax.experimental.pallas import tpu as pltpu

_D = 768
_E = 64
_K = 8
_LBW = 0.01

_BT = 2048


def _router_block(x_ref, w_ref, probs_ref, wts_ref, idx_ref, loss_ref,
                  acc_p, acc_c, *, nsteps, num_tokens):
    step = pl.program_id(0) * pl.num_programs(1) + pl.program_id(1)

    xb = x_ref[0]
    w = w_ref[...]
    logits = lax.dot_general(w, xb, (((1,), (1,)), ((), ())),
                             preferred_element_type=jnp.float32)

    m = jnp.max(logits, axis=0, keepdims=True)
    ex = jnp.exp(logits - m)
    s = jnp.sum(ex, axis=0, keepdims=True)
    probs = ex * (1.0 / s)
    probs_ref[0] = probs.T

    row = lax.broadcasted_iota(jnp.int32, (_E, _BT), 0)
    keys = lax.bitcast_convert_type(probs, jnp.int32)
    vals, idxs = [], []
    for _ in range(_K):
        mk = jnp.max(keys, axis=0, keepdims=True)
        ik = jnp.min(jnp.where(keys == mk, row, _E), axis=0, keepdims=True)
        keys = jnp.where(row == ik, -1, keys)
        vals.append(lax.bitcast_convert_type(mk, jnp.float32))
        idxs.append(ik.astype(jnp.float32))
    sel = (keys < 0).astype(jnp.float32)

    w8 = jnp.concatenate(vals, axis=0)
    w8 = w8 * (1.0 / jnp.sum(w8, axis=0, keepdims=True))
    wts_ref[0] = w8
    i8 = jnp.concatenate(idxs, axis=0)
    idx_ref[0] = i8.astype(jnp.int32)

    psum = jnp.sum(probs, axis=1, keepdims=True)
    csum = jnp.sum(sel, axis=1, keepdims=True)

    @pl.when(step == 0)
    def _init():
        acc_p[...] = psum
        acc_c[...] = csum

    @pl.when(step > 0)
    def _acc():
        acc_p[...] += psum
        acc_c[...] += csum

    @pl.when(step == nsteps - 1)
    def _fin():
        scale = (_LBW * _E) / (float(num_tokens) * float(num_tokens))
        loss_ref[...] = scale * jnp.sum(acc_p[...] * acc_c[...],
                                        keepdims=True).reshape(1, 1)


def kernel(x, W):
    B, S, D = x.shape
    T = B * S
    nsb = S // _BT
    nsteps = B * nsb
    body = functools.partial(_router_block, nsteps=nsteps, num_tokens=T)
    probs, wts, idx, loss = pl.pallas_call(
        body,
        grid=(B, nsb),
        in_specs=[
            pl.BlockSpec((1, _BT, D), lambda b, i: (b, i, 0)),
            pl.BlockSpec((_E, D), lambda b, i: (0, 0)),
        ],
        out_specs=[
            pl.BlockSpec((1, _BT, _E), lambda b, i: (b, i, 0)),
            pl.BlockSpec((1, _K, _BT), lambda b, i: (b, 0, i)),
            pl.BlockSpec((1, _K, _BT), lambda b, i: (b, 0, i)),
            pl.BlockSpec((1, 1), lambda b, i: (0, 0)),
        ],
        out_shape=[
            jax.ShapeDtypeStruct((B, S, _E), jnp.float32),
            jax.ShapeDtypeStruct((B, _K, S), jnp.float32),
            jax.ShapeDtypeStruct((B, _K, S), jnp.int32),
            jax.ShapeDtypeStruct((1, 1), jnp.float32),
        ],
        scratch_shapes=[
            pltpu.VMEM((_E, 1), jnp.float32),
            pltpu.VMEM((_E, 1), jnp.float32),
        ],
        compiler_params=pltpu.CompilerParams(
            dimension_semantics=("arbitrary", "arbitrary")),
    )(x, W)
    return (wts.transpose(0, 2, 1), idx.transpose(0, 2, 1), loss[0, 0], probs)


# speedup vs baseline: 4.1444x; 1.0746x over previous
import functools

import jax
import jax.numpy as jnp
from jax import lax
from jax.experimental import pallas as pl
from jax.experimental.pallas import tpu as pltpu

_D = 768
_E = 64
_K = 8
_LBW = 0.01

_BT = 4096


def _router_block(x_ref, w_ref, probs_ref, wts_ref, idx_ref, loss_ref,
                  acc_p, acc_c, *, nsteps, num_tokens):
    step = pl.program_id(0) * pl.num_programs(1) + pl.program_id(1)

    xb = x_ref[0]
    w = w_ref[...]
    logits = lax.dot_general(w, xb, (((1,), (1,)), ((), ())),
                             preferred_element_type=jnp.float32)

    m = jnp.max(logits, axis=0, keepdims=True)
    ex = jnp.exp(logits - m)
    s = jnp.sum(ex, axis=0, keepdims=True)
    probs = ex * (1.0 / s)
    probs_ref[0] = probs.T

    row = lax.broadcasted_iota(jnp.int32, (_E, _BT), 0)
    keys = lax.bitcast_convert_type(probs, jnp.int32)
    vals, idxs = [], []
    for _ in range(_K):
        mk = jnp.max(keys, axis=0, keepdims=True)
        ik = jnp.min(jnp.where(keys == mk, row, _E), axis=0, keepdims=True)
        keys = jnp.where(row == ik, -1, keys)
        vals.append(lax.bitcast_convert_type(mk, jnp.float32))
        idxs.append(ik.astype(jnp.float32))
    sel = (keys < 0).astype(jnp.float32)

    w8 = jnp.concatenate(vals, axis=0)
    w8 = w8 * (1.0 / jnp.sum(w8, axis=0, keepdims=True))
    wts_ref[0] = w8
    i8 = jnp.concatenate(idxs, axis=0)
    idx_ref[0] = i8.astype(jnp.int32)

    psum = jnp.sum(probs, axis=1, keepdims=True)
    csum = jnp.sum(sel, axis=1, keepdims=True)

    @pl.when(step == 0)
    def _init():
        acc_p[...] = psum
        acc_c[...] = csum

    @pl.when(step > 0)
    def _acc():
        acc_p[...] += psum
        acc_c[...] += csum

    @pl.when(step == nsteps - 1)
    def _fin():
        scale = (_LBW * _E) / (float(num_tokens) * float(num_tokens))
        loss_ref[...] = scale * jnp.sum(acc_p[...] * acc_c[...],
                                        keepdims=True).reshape(1, 1)


def kernel(x, W):
    B, S, D = x.shape
    T = B * S
    nsb = S // _BT
    nsteps = B * nsb
    body = functools.partial(_router_block, nsteps=nsteps, num_tokens=T)
    probs, wts, idx, loss = pl.pallas_call(
        body,
        grid=(B, nsb),
        in_specs=[
            pl.BlockSpec((1, _BT, D), lambda b, i: (b, i, 0)),
            pl.BlockSpec((_E, D), lambda b, i: (0, 0)),
        ],
        out_specs=[
            pl.BlockSpec((1, _BT, _E), lambda b, i: (b, i, 0)),
            pl.BlockSpec((1, _K, _BT), lambda b, i: (b, 0, i)),
            pl.BlockSpec((1, _K, _BT), lambda b, i: (b, 0, i)),
            pl.BlockSpec((1, 1), lambda b, i: (0, 0)),
        ],
        out_shape=[
            jax.ShapeDtypeStruct((B, S, _E), jnp.float32),
            jax.ShapeDtypeStruct((B, _K, S), jnp.float32),
            jax.ShapeDtypeStruct((B, _K, S), jnp.int32),
            jax.ShapeDtypeStruct((1, 1), jnp.float32),
        ],
        scratch_shapes=[
            pltpu.VMEM((_E, 1), jnp.float32),
            pltpu.VMEM((_E, 1), jnp.float32),
        ],
        compiler_params=pltpu.CompilerParams(
            dimension_semantics=("arbitrary", "arbitrary")),
    )(x, W)
    return (wts.transpose(0, 2, 1), idx.transpose(0, 2, 1), loss[0, 0], probs)
